# baseline (speedup 1.0000x reference)
import functools

import jax
import jax.numpy as jnp
from jax import lax
from jax.experimental import pallas as pl
from jax.experimental.pallas import tpu as pltpu
from jax.experimental.pallas import tpu_sc as plsc

N = 10000
E = 320000
D_IN = 128
D_PAD = 16
VAR_EPS = 1e-4

NC = 2
NS = 16
NW = NC * NS
E_PAD = 327680
EPW = E_PAD // NW
CH = 2048
K = CH // 128
CHUNKS = EPW // CH
NB_I = 3
NB_R = 2
N_PAD = 10240
ROWS_PER_SUB = N_PAD // NS

_MESH = plsc.VectorSubcoreMesh(core_axis_name="c", subcore_axis_name="s")


@functools.partial(
    pl.kernel,
    out_type=jax.ShapeDtypeStruct((NC, N_PAD, D_PAD), jnp.float32),
    mesh=_MESH,
    scratch_types=[
        pltpu.VMEM((NB_I, K, 128), jnp.int32),
        pltpu.VMEM((NB_I, K, 128), jnp.int32),
        pltpu.VMEM((NB_I, CH), jnp.float32),
        pltpu.VMEM((NB_R, CH, D_PAD), jnp.float32),
        pltpu.VMEM_SHARED((N_PAD, D_PAD), jnp.float32),
        pltpu.SemaphoreType.DMA,
        pltpu.SemaphoreType.DMA,
        pltpu.SemaphoreType.DMA,
        pltpu.SemaphoreType.DMA,
        pltpu.SemaphoreType.DMA,
        pltpu.SemaphoreType.DMA,
        pltpu.SemaphoreType.DMA,
        pltpu.SemaphoreType.DMA,
    ],
    compiler_params=pltpu.CompilerParams(use_tc_tiling_on_sc=False),
)
def _sc_spmm(support_hbm, src_hbm, dst_hbm, w_hbm, zero_hbm, out_hbm,
             src_v, dst_v, w_v, rows_v, qm_sh,
             sem_z, sem_i0, sem_i1, sem_i2, sem_g0, sem_g1, sem_s0, sem_s1):
    sem_i = [sem_i0, sem_i1, sem_i2]
    sem_g = [sem_g0, sem_g1]
    sem_s = [sem_s0, sem_s1]
    cid = lax.axis_index("c")
    sid = lax.axis_index("s")
    wid = cid * NS + sid
    r0 = pl.multiple_of(sid * ROWS_PER_SUB, 8)

    def fire_idx(c):
        b = c % NB_I
        base_e = pl.multiple_of(wid * EPW + c * CH, 8)
        base_r = pl.multiple_of(wid * (EPW // 128) + c * K, 8)
        return [
            pltpu.async_copy(src_hbm.at[pl.ds(base_r, K)], src_v.at[b],
                             sem_i[b]),
            pltpu.async_copy(dst_hbm.at[pl.ds(base_r, K)], dst_v.at[b],
                             sem_i[b]),
            pltpu.async_copy(w_hbm.at[pl.ds(base_e, CH)], w_v.at[b],
                             sem_i[b]),
        ]

    def fire_gather(c):
        bi, br = c % NB_I, c % NB_R
        return [
            pltpu.async_copy(support_hbm.at[src_v.at[bi, j]],
                             rows_v.at[br, pl.ds(j * 128, 128)], sem_g[br])
            for j in range(K)
        ]

    def fire_scatter(c):
        bi, br = c % NB_I, c % NB_R
        return [
            pltpu.async_copy(rows_v.at[br, pl.ds(j * 128, 128)],
                             qm_sh.at[dst_v.at[bi, j]], sem_s[br], add=True)
            for j in range(K)
        ]

    def scale(c):
        bi, br = c % NB_I, c % NB_R

        @pl.loop(0, CH, step=16)
        def _(g):
            wv = w_v[bi, pl.ds(g, 16)]
            for i in range(16):
                ws = jnp.take_along_axis(
                    wv, jnp.full((16,), i, jnp.int32), axis=0)
                rows_v[br, g + i, :] = rows_v[br, g + i, :] * ws

    def drain(descs):
        for d in descs:
            d.wait()

    di = {0: fire_idx(0), 1: fire_idx(1)}
    zcp = pltpu.async_copy(zero_hbm.at[pl.ds(r0, ROWS_PER_SUB)],
                           qm_sh.at[pl.ds(r0, ROWS_PER_SUB)], sem_z)
    zcp.wait()
    plsc.subcore_barrier()
    drain(di[0])
    dg = {0: fire_gather(0)}
    ds = {}

    for c in range(CHUNKS):
        drain(dg[c])
        if c + 1 < CHUNKS:
            drain(di[c + 1])
            if c >= 1:
                drain(ds[c - 1])
            dg[c + 1] = fire_gather(c + 1)
        scale(c)
        if c + 2 < CHUNKS:
            di[c + 2] = fire_idx(c + 2)
        ds[c] = fire_scatter(c)

    drain(ds[CHUNKS - 2])
    drain(ds[CHUNKS - 1])
    plsc.subcore_barrier()
    pltpu.sync_copy(qm_sh.at[pl.ds(r0, ROWS_PER_SUB)],
                    out_hbm.at[cid, pl.ds(r0, ROWS_PER_SUB)])


def _support_body(x_ref, w_ref, o_ref):
    o_ref[...] = jnp.dot(x_ref[...], w_ref[...],
                         preferred_element_type=jnp.float32)


def _var_body(x_ref, w_ref, b_ref, o_ref):
    z = lax.dot_general(w_ref[...], x_ref[...],
                        dimension_numbers=(((1,), (1,)), ((), ())),
                        precision=lax.Precision.HIGHEST,
                        preferred_element_type=jnp.float32)
    z = z + b_ref[:, 0:1]
    o_ref[...] = jnp.sqrt(jnp.exp(z[:D_OUT, :]) + VAR_EPS)


def _combine_body(p0_ref, p1_ref, eye_ref, s_ref, e_ref, qm_ref, lat_ref):
    qm16 = p0_ref[0] + p1_ref[0]
    qmt = lax.dot_general(eye_ref[...], qm16,
                          dimension_numbers=(((1,), (1,)), ((), ())),
                          precision=lax.Precision.HIGHEST,
                          preferred_element_type=jnp.float32)
    qm = qmt[:D_OUT, :N]
    qm_ref[...] = qm
    lat_ref[...] = qm + s_ref[...] * e_ref[...]


D_OUT = 10
_BN = 1000


def kernel(x, edge_index, edge_weight, eps, W_gcn, W_var, b_var):
    src = edge_index[1].astype(jnp.int32)
    dst = edge_index[0].astype(jnp.int32)
    pad = E_PAD - E
    pad_src = jnp.arange(pad, dtype=jnp.int32) % N
    src = jnp.concatenate([src, pad_src]).reshape(E_PAD // 128, 128)
    pad_dst = N + (jnp.arange(pad, dtype=jnp.int32) % (N_PAD - N))
    dst = jnp.concatenate([dst, pad_dst]).reshape(E_PAD // 128, 128)
    w = jnp.pad(edge_weight.astype(jnp.float32), (0, pad))
    Wg = jnp.pad(W_gcn, ((0, 0), (0, D_PAD - W_gcn.shape[1])))
    Wvt = jnp.pad(W_var, ((0, D_PAD - D_OUT), (0, 0)))
    bbt = jnp.broadcast_to(jnp.pad(b_var, (0, D_PAD - D_OUT))[:, None],
                           (D_PAD, 128))
    eps_t = jnp.transpose(eps)
    eye16 = jnp.eye(D_PAD, dtype=jnp.float32)
    zeros = jnp.zeros((N_PAD, D_PAD), jnp.float32)

    support = pl.pallas_call(
        _support_body,
        grid=(N // _BN,),
        in_specs=[
            pl.BlockSpec((_BN, D_IN), lambda i: (i, 0)),
            pl.BlockSpec((D_IN, D_PAD), lambda i: (0, 0)),
        ],
        out_specs=pl.BlockSpec((_BN, D_PAD), lambda i: (i, 0)),
        out_shape=jax.ShapeDtypeStruct((N, D_PAD), jnp.float32),
    )(x, Wg)

    partials = _sc_spmm(support, src, dst, w, zeros)

    scale_t = pl.pallas_call(
        _var_body,
        grid=(1,),
        in_specs=[
            pl.BlockSpec((N, D_IN), lambda i: (0, 0)),
            pl.BlockSpec((D_PAD, D_IN), lambda i: (0, 0)),
            pl.BlockSpec((D_PAD, 128), lambda i: (0, 0)),
        ],
        out_specs=pl.BlockSpec((D_OUT, N), lambda i: (0, 0)),
        out_shape=jax.ShapeDtypeStruct((D_OUT, N), jnp.float32),
    )(x, Wvt, bbt)

    q_m_t, latent_t = pl.pallas_call(
        _combine_body,
        grid=(1,),
        in_specs=[
            pl.BlockSpec((1, N_PAD, D_PAD), lambda i: (0, 0, 0)),
            pl.BlockSpec((1, N_PAD, D_PAD), lambda i: (1, 0, 0)),
            pl.BlockSpec((D_PAD, D_PAD), lambda i: (0, 0)),
            pl.BlockSpec((D_OUT, N), lambda i: (0, 0)),
            pl.BlockSpec((D_OUT, N), lambda i: (0, 0)),
        ],
        out_specs=[
            pl.BlockSpec((D_OUT, N), lambda i: (0, 0)),
            pl.BlockSpec((D_OUT, N), lambda i: (0, 0)),
        ],
        out_shape=[
            jax.ShapeDtypeStruct((D_OUT, N), jnp.float32),
            jax.ShapeDtypeStruct((D_OUT, N), jnp.float32),
        ],
    )(partials, partials, eye16, scale_t, eps_t)

    return (jnp.transpose(q_m_t), jnp.transpose(scale_t),
            jnp.transpose(latent_t))


# speedup vs baseline: 1.2959x; 1.2959x over previous
import functools

import jax
import jax.numpy as jnp
from jax import lax
from jax.experimental import pallas as pl
from jax.experimental.pallas import tpu as pltpu
from jax.experimental.pallas import tpu_sc as plsc

N = 10000
E = 320000
D_IN = 128
D_PAD = 16
VAR_EPS = 1e-4

NC = 2
NS = 16
NW = NC * NS
G = E // 128
GPW = G // NW
GX = G - GPW * NW
K = 8
CHUNKS = 10
K_LAST = GPW - (CHUNKS - 1) * K
NB_I = 3
NB_R = 2
N_PAD = 10240
ROWS_PER_SUB = N_PAD // NS

_MESH = plsc.VectorSubcoreMesh(core_axis_name="c", subcore_axis_name="s")


@functools.partial(
    pl.kernel,
    out_type=jax.ShapeDtypeStruct((NC, N_PAD, D_PAD), jnp.float32),
    mesh=_MESH,
    scratch_types=[
        pltpu.VMEM((NB_I, K, 2, 128), jnp.int32),
        pltpu.VMEM((NB_I, K * 128), jnp.float32),
        pltpu.VMEM((NB_R, K * 128, D_PAD), jnp.float32),
        pltpu.VMEM_SHARED((N_PAD, D_PAD), jnp.float32),
        pltpu.SemaphoreType.DMA,
        pltpu.SemaphoreType.DMA,
        pltpu.SemaphoreType.DMA,
        pltpu.SemaphoreType.DMA,
        pltpu.SemaphoreType.DMA,
        pltpu.SemaphoreType.DMA,
        pltpu.SemaphoreType.DMA,
        pltpu.SemaphoreType.DMA,
    ],
    compiler_params=pltpu.CompilerParams(use_tc_tiling_on_sc=False),
)
def _sc_spmm(support_hbm, ei_hbm, w_hbm, zero_hbm, out_hbm,
             idx_v, w_v, rows_v, qm_sh,
             sem_z, sem_i0, sem_i1, sem_i2, sem_g0, sem_g1, sem_s0, sem_s1):
    sem_i = [sem_i0, sem_i1, sem_i2]
    sem_g = [sem_g0, sem_g1]
    sem_s = [sem_s0, sem_s1]
    cid = lax.axis_index("c")
    sid = lax.axis_index("s")
    wid = cid * NS + sid
    g0 = wid * GPW + jnp.minimum(wid, GX)
    r0 = pl.multiple_of(sid * ROWS_PER_SUB, 8)

    def ksize(c):
        return K if c < CHUNKS - 1 else K_LAST

    def fire_idx(c):
        b = c % NB_I
        gb = g0 + c * K
        k = ksize(c)
        return [
            pltpu.async_copy(ei_hbm.at[pl.ds(gb, k)],
                             idx_v.at[b, pl.ds(0, k)], sem_i[b]),
            pltpu.async_copy(w_hbm.at[pl.ds(pl.multiple_of(gb * 128, 128),
                                            k * 128)],
                             w_v.at[b, pl.ds(0, k * 128)], sem_i[b]),
        ]

    def fire_gather(c):
        bi, br = c % NB_I, c % NB_R
        return [
            pltpu.async_copy(support_hbm.at[idx_v.at[bi, j, 1]],
                             rows_v.at[br, pl.ds(j * 128, 128)], sem_g[br])
            for j in range(ksize(c))
        ]

    def fire_scatter(c):
        bi, br = c % NB_I, c % NB_R
        return [
            pltpu.async_copy(rows_v.at[br, pl.ds(j * 128, 128)],
                             qm_sh.at[idx_v.at[bi, j, 0]], sem_s[br], add=True)
            for j in range(ksize(c))
        ]

    def scale(c):
        bi, br = c % NB_I, c % NB_R

        @pl.loop(0, ksize(c) * 128, step=16)
        def _(g):
            wv = w_v[bi, pl.ds(g, 16)]
            for i in range(16):
                ws = jnp.take_along_axis(
                    wv, jnp.full((16,), i, jnp.int32), axis=0)
                rows_v[br, g + i, :] = rows_v[br, g + i, :] * ws

    def drain(descs):
        for d in descs:
            d.wait()

    di = {0: fire_idx(0), 1: fire_idx(1)}
    zcp = pltpu.async_copy(zero_hbm.at[pl.ds(r0, ROWS_PER_SUB)],
                           qm_sh.at[pl.ds(r0, ROWS_PER_SUB)], sem_z)
    zcp.wait()
    plsc.subcore_barrier()
    drain(di[0])
    dg = {0: fire_gather(0)}
    ds = {}

    for c in range(CHUNKS):
        drain(dg[c])
        if c + 1 < CHUNKS:
            drain(di[c + 1])
            if c >= 1:
                drain(ds[c - 1])
            dg[c + 1] = fire_gather(c + 1)
        scale(c)
        if c + 2 < CHUNKS:
            di[c + 2] = fire_idx(c + 2)
        ds[c] = fire_scatter(c)

    drain(ds[CHUNKS - 2])
    drain(ds[CHUNKS - 1])

    @pl.when(wid < GX)
    def _():
        gx = g0 + GPW
        pltpu.sync_copy(ei_hbm.at[pl.ds(gx, 1)], idx_v.at[0, pl.ds(0, 1)])
        pltpu.sync_copy(w_hbm.at[pl.ds(gx * 128, 128)],
                        w_v.at[0, pl.ds(0, 128)])
        pltpu.sync_copy(support_hbm.at[idx_v.at[0, 0, 1]],
                        rows_v.at[0, pl.ds(0, 128)])

        @pl.loop(0, 128, step=16)
        def _(g):
            wv = w_v[0, pl.ds(g, 16)]
            for i in range(16):
                ws = jnp.take_along_axis(
                    wv, jnp.full((16,), i, jnp.int32), axis=0)
                rows_v[0, g + i, :] = rows_v[0, g + i, :] * ws

        pltpu.sync_copy(rows_v.at[0, pl.ds(0, 128)],
                        qm_sh.at[idx_v.at[0, 0, 0]], add=True)

    plsc.subcore_barrier()
    pltpu.sync_copy(qm_sh.at[pl.ds(r0, ROWS_PER_SUB)],
                    out_hbm.at[cid, pl.ds(r0, ROWS_PER_SUB)])


def _support_body(x_ref, w_ref, o_ref):
    parts = [
        jnp.dot(x_ref[:, r, :], w_ref[...], preferred_element_type=jnp.float32)
        for r in range(8)
    ]
    o_ref[...] = jnp.concatenate(parts, axis=1)


def _var_body(x_ref, w_ref, b_ref, o_ref):
    z = lax.dot_general(w_ref[...], x_ref[...],
                        dimension_numbers=(((1,), (1,)), ((), ())),
                        precision=lax.Precision.HIGHEST,
                        preferred_element_type=jnp.float32)
    z = z + b_ref[:, 0:1]
    o_ref[...] = jnp.sqrt(jnp.exp(z[:D_OUT, :]) + VAR_EPS)


def _combine_body(p0_ref, p1_ref, eye_ref, s_ref, e_ref, qm_ref, lat_ref):
    qm16 = p0_ref[0] + p1_ref[0]
    qmt = lax.dot_general(eye_ref[...], qm16,
                          dimension_numbers=(((1,), (1,)), ((), ())),
                          precision=lax.Precision.HIGHEST,
                          preferred_element_type=jnp.float32)
    qm = qmt[:D_OUT, :N]
    qm_ref[...] = qm
    lat_ref[...] = qm + s_ref[...] * e_ref[...]


D_OUT = 10
_BN = 1000


def kernel(x, edge_index, edge_weight, eps, W_gcn, W_var, b_var):
    ei3 = jnp.transpose(edge_index.astype(jnp.int32).reshape(2, G, 128),
                        (1, 0, 2))
    w = edge_weight
    Wg = jnp.pad(W_gcn, ((0, 0), (0, D_PAD - W_gcn.shape[1])))
    Wvt = jnp.pad(W_var, ((0, D_PAD - D_OUT), (0, 0)))
    bbt = jnp.broadcast_to(jnp.pad(b_var, (0, D_PAD - D_OUT))[:, None],
                           (D_PAD, 128))
    eps_t = jnp.transpose(eps)
    eye16 = jnp.eye(D_PAD, dtype=jnp.float32)
    zeros = jnp.zeros((N_PAD, D_PAD), jnp.float32)

    support = pl.pallas_call(
        _support_body,
        grid=(1,),
        in_specs=[
            pl.BlockSpec((N // 8, 8, D_IN), lambda i: (0, 0, 0)),
            pl.BlockSpec((D_IN, D_PAD), lambda i: (0, 0)),
        ],
        out_specs=pl.BlockSpec((N // 8, 128), lambda i: (0, 0)),
        out_shape=jax.ShapeDtypeStruct((N // 8, 128), jnp.float32),
    )(x.reshape(N // 8, 8, D_IN), Wg)

    partials = _sc_spmm(support.reshape(N, D_PAD), ei3, w, zeros)

    scale_t = pl.pallas_call(
        _var_body,
        grid=(1,),
        in_specs=[
            pl.BlockSpec((N, D_IN), lambda i: (0, 0)),
            pl.BlockSpec((D_PAD, D_IN), lambda i: (0, 0)),
            pl.BlockSpec((D_PAD, 128), lambda i: (0, 0)),
        ],
        out_specs=pl.BlockSpec((D_OUT, N), lambda i: (0, 0)),
        out_shape=jax.ShapeDtypeStruct((D_OUT, N), jnp.float32),
    )(x, Wvt, bbt)

    q_m_t, latent_t = pl.pallas_call(
        _combine_body,
        grid=(1,),
        in_specs=[
            pl.BlockSpec((1, N_PAD, D_PAD), lambda i: (0, 0, 0)),
            pl.BlockSpec((1, N_PAD, D_PAD), lambda i: (1, 0, 0)),
            pl.BlockSpec((D_PAD, D_PAD), lambda i: (0, 0)),
            pl.BlockSpec((D_OUT, N), lambda i: (0, 0)),
            pl.BlockSpec((D_OUT, N), lambda i: (0, 0)),
        ],
        out_specs=[
            pl.BlockSpec((D_OUT, N), lambda i: (0, 0)),
            pl.BlockSpec((D_OUT, N), lambda i: (0, 0)),
        ],
        out_shape=[
            jax.ShapeDtypeStruct((D_OUT, N), jnp.float32),
            jax.ShapeDtypeStruct((D_OUT, N), jnp.float32),
        ],
    )(partials, partials, eye16, scale_t, eps_t)

    return (jnp.transpose(q_m_t), jnp.transpose(scale_t),
            jnp.transpose(latent_t))
